# baseline (speedup 1.0000x reference)
import functools

import jax
import jax.numpy as jnp
from jax import lax
from jax.experimental import pallas as pl
from jax.experimental.pallas import tpu as pltpu
from jax.experimental.pallas import tpu_sc as plsc

NUM_CATS = 100000
DIM = 128
ROWS = 4096
SEQ = 200
B = ROWS * SEQ

_INFO = plsc.get_sparse_core_info()
NC = _INFO.num_cores
NS = _INFO.num_subcores
NW = NC * NS
GROUP = 64
B_PER_W = B // NW
G = B_PER_W // GROUP
G2 = B_PER_W // 128
NBUF = 10
LOOKAHEAD = 6
NITER = G // NBUF
assert G % NBUF == 0 and LOOKAHEAD < NBUF and NBUF % 2 == 0


def _idx_slice(idx_v, jj, b):
    row = jj * (NBUF // 2) + b // 2
    return idx_v.at[row, pl.ds((b % 2) * GROUP, GROUP)]


@functools.partial(
    pl.kernel,
    mesh=plsc.VectorSubcoreMesh(core_axis_name="c", subcore_axis_name="s"),
    out_type=jax.ShapeDtypeStruct((B, DIM), jnp.float32),
    scratch_types=(
        [pltpu.VMEM((G2, 128), jnp.int32)]
        + [pltpu.VMEM((GROUP, DIM), jnp.float32) for _ in range(NBUF)]
        + [pltpu.SemaphoreType.DMA for _ in range(2 * NBUF)]
    ),
)
def _embed_gather(table_hbm, idx_hbm, out_hbm, idx_v, *rest):
    bufs = rest[:NBUF]
    gsems = rest[NBUF:2 * NBUF]
    osems = rest[2 * NBUF:]
    wid = lax.axis_index("s") * NC + lax.axis_index("c")
    base = wid * B_PER_W
    pltpu.sync_copy(idx_hbm.at[wid], idx_v)

    for b in range(LOOKAHEAD):
        pltpu.async_copy(table_hbm.at[_idx_slice(idx_v, 0, b)], bufs[b],
                         gsems[b])

    def step(jj, carry):
        for b in range(NBUF):
            j = jj * NBUF + b
            bn = (b + LOOKAHEAD) % NBUF
            pltpu.make_async_copy(
                table_hbm.at[_idx_slice(idx_v, 0, 0)], bufs[b], gsems[b]
            ).wait()
            pltpu.async_copy(
                bufs[b], out_hbm.at[pl.ds(base + j * GROUP, GROUP)], osems[b]
            )

            def drain_nbr():
                pltpu.make_async_copy(
                    bufs[bn], out_hbm.at[pl.ds(0, GROUP)], osems[bn]
                ).wait()

            def start_next():
                bl = (b + LOOKAHEAD) % NBUF
                jn = jj + (b + LOOKAHEAD) // NBUF
                pltpu.async_copy(
                    table_hbm.at[_idx_slice(idx_v, jn, bl)], bufs[bn],
                    gsems[bn]
                )

            if b < NBUF - LOOKAHEAD:
                pl.when(jj >= 1)(drain_nbr)
                start_next()
            else:
                drain_nbr()
                pl.when(jj < NITER - 1)(start_next)
        return carry

    lax.fori_loop(0, NITER, step, 0)
    for j in range(G - (NBUF - LOOKAHEAD), G):
        pltpu.make_async_copy(
            bufs[j % NBUF], out_hbm.at[pl.ds(0, GROUP)], osems[j % NBUF]
        ).wait()


def kernel(cat_ids, table):
    idx3 = cat_ids.reshape(NW, G2, 128)
    out = _embed_gather(table, idx3)
    return out.reshape(ROWS, SEQ, DIM)
